# Initial kernel scaffold; baseline (speedup 1.0000x reference)
import jax
import jax.numpy as jnp
from jax.experimental import pallas as pl


def kernel(user, item_p, item_n, mask, users_int_w, users_pop_w, items_int_w, items_pop_w):
    raise NotImplementedError("write your pallas kernel here")


# speedup vs baseline: 1.1467x; 1.1467x over previous
import jax
import jax.numpy as jnp
from jax import lax
from jax.experimental import pallas as pl
from jax.experimental.pallas import tpu as pltpu
from jax.experimental.pallas import tpu_sc as plsc

D = 16
LN = 16
NC, NS = 2, 16
NW = NC * NS
N_OCC = 16384 * 20
IDX_ROWS = N_OCC // 128
CHUNK = 512
SUBS = N_OCC // NW // CHUNK
CH_ROWS = CHUNK // 128
NTAB = 1 << 20
ZB = 4096
ROWS_PER_TILE = IDX_ROWS // NS
RB = 8
BLK = 4096
TGRID = N_OCC // BLK

_mesh = plsc.VectorSubcoreMesh(core_axis_name="c", subcore_axis_name="s")


def _gather_body(uiw, upw, iiw, ipw, uidx, pidx, nidx,
                 o_ui, o_up, o_pi, o_pp, o_ni, o_np,
                 idx_u, idx_p, idx_n,
                 r_ui, r_up, r_pi, r_pp, r_ni, r_np, sem):
  c = lax.axis_index("c")
  s = lax.axis_index("s")
  wid = c * NS + s

  def sub(k, _):
    row0 = wid * (SUBS * CH_ROWS) + k * CH_ROWS
    pltpu.sync_copy(uidx.at[pl.ds(row0, CH_ROWS)], idx_u)
    pltpu.sync_copy(pidx.at[pl.ds(row0, CH_ROWS)], idx_p)
    pltpu.sync_copy(nidx.at[pl.ds(row0, CH_ROWS)], idx_n)
    cps = []
    for j in range(CH_ROWS):
      sl = pl.ds(j * 128, 128)
      cps.append(pltpu.async_copy(uiw.at[idx_u.at[j]], r_ui.at[sl], sem))
      cps.append(pltpu.async_copy(upw.at[idx_u.at[j]], r_up.at[sl], sem))
      cps.append(pltpu.async_copy(iiw.at[idx_p.at[j]], r_pi.at[sl], sem))
      cps.append(pltpu.async_copy(ipw.at[idx_p.at[j]], r_pp.at[sl], sem))
      cps.append(pltpu.async_copy(iiw.at[idx_n.at[j]], r_ni.at[sl], sem))
      cps.append(pltpu.async_copy(ipw.at[idx_n.at[j]], r_np.at[sl], sem))
    for cp in cps:
      cp.wait()
    base = wid * (SUBS * CHUNK) + k * CHUNK
    osl = pl.ds(base, CHUNK)
    pltpu.sync_copy(r_ui, o_ui.at[osl])
    pltpu.sync_copy(r_up, o_up.at[osl])
    pltpu.sync_copy(r_pi, o_pi.at[osl])
    pltpu.sync_copy(r_pp, o_pp.at[osl])
    pltpu.sync_copy(r_ni, o_ni.at[osl])
    pltpu.sync_copy(r_np, o_np.at[osl])
    return 0

  lax.fori_loop(0, SUBS, sub, 0)


_gather = pl.kernel(
    _gather_body,
    out_type=tuple(jax.ShapeDtypeStruct((N_OCC, D), jnp.float32)
                   for _ in range(6)),
    mesh=_mesh,
    compiler_params=pltpu.CompilerParams(use_tc_tiling_on_sc=False),
    scratch_types=[
        pltpu.VMEM((CH_ROWS, 128), jnp.int32),
        pltpu.VMEM((CH_ROWS, 128), jnp.int32),
        pltpu.VMEM((CH_ROWS, 128), jnp.int32),
        pltpu.VMEM((CHUNK, D), jnp.float32),
        pltpu.VMEM((CHUNK, D), jnp.float32),
        pltpu.VMEM((CHUNK, D), jnp.float32),
        pltpu.VMEM((CHUNK, D), jnp.float32),
        pltpu.VMEM((CHUNK, D), jnp.float32),
        pltpu.VMEM((CHUNK, D), jnp.float32),
        pltpu.SemaphoreType.DMA,
    ],
)


def _disc_body(uidx, pidx, nidx, ru, rp, rn,
               counts, zbuf, onesb, idxb, cntb, recb):
  c = lax.axis_index("c")
  s = lax.axis_index("s")

  def fill(i, _):
    zbuf[pl.ds(i * LN, LN)] = jnp.zeros((LN,), jnp.float32)
    return 0

  lax.fori_loop(0, ZB // LN, fill, 0)

  def ones_fill(i, _):
    onesb[pl.ds(i * LN, LN)] = jnp.ones((LN,), jnp.float32)
    return 0

  lax.fori_loop(0, 128 // LN, ones_fill, 0)

  def zero(i, _):
    pltpu.sync_copy(zbuf, counts.at[pl.ds(s * (NTAB // NS) + i * ZB, ZB)])
    return 0

  lax.fori_loop(0, NTAB // NS // ZB, zero, 0)
  plsc.subcore_barrier()

  def scat_rows(idx_hbm, r0):
    def batch(b, _):
      pltpu.sync_copy(idx_hbm.at[pl.ds(r0 + b * RB, RB)], idxb)
      for j in range(RB):
        pltpu.sync_copy(onesb, counts.at[idxb.at[j]], add=True)
      return 0

    lax.fori_loop(0, ROWS_PER_TILE // RB, batch, 0)

  @pl.when(c == 0)
  def _():
    scat_rows(uidx, s * ROWS_PER_TILE)

  @pl.when(c == 1)
  def _():
    scat_rows(pidx, s * ROWS_PER_TILE)
    scat_rows(nidx, s * ROWS_PER_TILE)

  plsc.subcore_barrier()

  def rec_rows(idx_hbm, out_hbm, r0):
    def batch(b, _):
      pltpu.sync_copy(idx_hbm.at[pl.ds(r0 + b * RB, RB)], idxb)
      for j in range(RB):
        pltpu.sync_copy(counts.at[idxb.at[j]], cntb.at[pl.ds(j * 128, 128)])

      def vec(k, _):
        o = pl.ds(k * LN, LN)
        recb[o] = 1.0 / cntb[o]
        return 0

      lax.fori_loop(0, RB * 128 // LN, vec, 0)
      pltpu.sync_copy(recb, out_hbm.at[pl.ds((r0 + b * RB) * 128, RB * 128)])
      return 0

    lax.fori_loop(0, ROWS_PER_TILE // RB, batch, 0)

  @pl.when(c == 0)
  def _():
    rec_rows(uidx, ru, s * ROWS_PER_TILE)

  @pl.when(c == 1)
  def _():
    rec_rows(pidx, rp, s * ROWS_PER_TILE)
    rec_rows(nidx, rn, s * ROWS_PER_TILE)


_disc = pl.kernel(
    _disc_body,
    out_type=tuple(jax.ShapeDtypeStruct((N_OCC,), jnp.float32)
                   for _ in range(3)),
    mesh=_mesh,
    compiler_params=pltpu.CompilerParams(use_tc_tiling_on_sc=False),
    scratch_types=[
        pltpu.VMEM_SHARED((NTAB,), jnp.float32),
        pltpu.VMEM((ZB,), jnp.float32),
        pltpu.VMEM((128,), jnp.float32),
        pltpu.VMEM((RB, 128), jnp.int32),
        pltpu.VMEM((RB * 128,), jnp.float32),
        pltpu.VMEM((RB * 128,), jnp.float32),
    ],
)


def _loss_body(ui_ref, up_ref, pi_ref, pp_ref, ni_ref, nq_ref,
               m_ref, ru_ref, rp_ref, rn_ref, out_ref, acc):
  k = pl.program_id(0)

  @pl.when(k == 0)
  def _():
    for i in range(8):
      acc[i] = 0.0

  ui = ui_ref[...]
  up = up_ref[...]
  pi = pi_ref[...]
  pp = pp_ref[...]
  ni = ni_ref[...]
  nq = nq_ref[...]
  m = m_ref[...].reshape(-1)
  ru = ru_ref[...].reshape(-1)
  rp = rp_ref[...].reshape(-1)
  rn = rn_ref[...].reshape(-1)

  a = jnp.sum(ui * (pi - ni), axis=1)
  b = jnp.sum(up * (pp - nq), axis=1)
  eps = 1e-10

  def lg(x):
    return jnp.log(jax.nn.sigmoid(x) + eps)

  s_int = jnp.sum(m * lg(a))
  s_pop = jnp.sum(m * lg(-b) + (1.0 - m) * lg(b))
  s_tot = jnp.sum(lg(a + b))

  squ = jnp.sum((ui - up) ** 2, axis=1)
  sqp = jnp.sum((pi - pp) ** 2, axis=1)
  sqn = jnp.sum((ni - nq) ** 2, axis=1)
  t_u = jnp.sum(squ * ru)
  n_u = jnp.sum(ru)
  t_i = jnp.sum(sqp * rp + sqn * rn)
  n_i = jnp.sum(rp + rn)

  acc[0] += s_int
  acc[1] += s_pop
  acc[2] += s_tot
  acc[3] += t_u
  acc[4] += n_u
  acc[5] += t_i
  acc[6] += n_i

  @pl.when(k == TGRID - 1)
  def _():
    n = float(N_OCC)
    loss_int = -acc[0] / n
    loss_pop = -acc[1] / n
    loss_total = -acc[2] / n
    disc = acc[3] / (acc[4] * D) + acc[5] / (acc[6] * D)
    out_ref[0, 0] = (0.1 * loss_int + 0.1 * loss_pop + loss_total
                     - 0.01 * disc)


_loss = pl.pallas_call(
    _loss_body,
    grid=(TGRID,),
    in_specs=[pl.BlockSpec((BLK, D), lambda k: (k, 0)) for _ in range(6)]
    + [pl.BlockSpec((BLK // 128, 128), lambda k: (k, 0)) for _ in range(4)],
    out_specs=pl.BlockSpec((1, 1), lambda k: (0, 0),
                           memory_space=pltpu.SMEM),
    out_shape=jax.ShapeDtypeStruct((1, 1), jnp.float32),
    scratch_shapes=[pltpu.SMEM((8,), jnp.float32)],
    compiler_params=pltpu.CompilerParams(
        dimension_semantics=("arbitrary",)),
)


@jax.jit
def kernel(user, item_p, item_n, mask, users_int_w, users_pop_w,
           items_int_w, items_pop_w):
  u = user.astype(jnp.int32).reshape(IDX_ROWS, 128)
  p = item_p.astype(jnp.int32).reshape(IDX_ROWS, 128)
  q = item_n.astype(jnp.int32).reshape(IDX_ROWS, 128)
  rows = _gather(users_int_w, users_pop_w, items_int_w, items_pop_w, u, p, q)
  ru, rp, rn = _disc(u, p, q)
  maskf = mask.reshape(-1).astype(jnp.float32).reshape(IDX_ROWS, 128)
  loss = _loss(*rows, maskf,
               ru.reshape(IDX_ROWS, 128), rp.reshape(IDX_ROWS, 128),
               rn.reshape(IDX_ROWS, 128))
  return loss[0, 0]


# speedup vs baseline: 1.9698x; 1.7178x over previous
import jax
import jax.numpy as jnp
from jax import lax
from jax.experimental import pallas as pl
from jax.experimental.pallas import tpu as pltpu
from jax.experimental.pallas import tpu_sc as plsc

D = 16
LN = 16
NC, NS = 2, 16
NW = NC * NS
N_OCC = 16384 * 20
IDX_ROWS = N_OCC // 128
CHUNK = 512
SUBS = N_OCC // NW // CHUNK
CH_ROWS = CHUNK // 128
NTAB = 1 << 20
ZB = 4096
ROWS_PER_TILE = IDX_ROWS // NS
RB = 8
BLK = 4096
TGRID = N_OCC // BLK

_mesh = plsc.VectorSubcoreMesh(core_axis_name="c", subcore_axis_name="s")


def _gather_body(uiw, upw, iiw, ipw, uidx, pidx, nidx,
                 o_ui, o_up, o_pi, o_pp, o_ni, o_np,
                 idx_u, idx_p, idx_n,
                 r_ui, r_up, r_pi, r_pp, r_ni, r_np, sem):
  c = lax.axis_index("c")
  s = lax.axis_index("s")
  wid = c * NS + s

  def sub(k, _):
    row0 = wid * (SUBS * CH_ROWS) + k * CH_ROWS
    pltpu.sync_copy(uidx.at[pl.ds(row0, CH_ROWS)], idx_u)
    pltpu.sync_copy(pidx.at[pl.ds(row0, CH_ROWS)], idx_p)
    pltpu.sync_copy(nidx.at[pl.ds(row0, CH_ROWS)], idx_n)
    cps = []
    for j in range(CH_ROWS):
      sl = pl.ds(j * 128, 128)
      cps.append(pltpu.async_copy(uiw.at[idx_u.at[j]], r_ui.at[sl], sem))
      cps.append(pltpu.async_copy(upw.at[idx_u.at[j]], r_up.at[sl], sem))
      cps.append(pltpu.async_copy(iiw.at[idx_p.at[j]], r_pi.at[sl], sem))
      cps.append(pltpu.async_copy(ipw.at[idx_p.at[j]], r_pp.at[sl], sem))
      cps.append(pltpu.async_copy(iiw.at[idx_n.at[j]], r_ni.at[sl], sem))
      cps.append(pltpu.async_copy(ipw.at[idx_n.at[j]], r_np.at[sl], sem))
    for cp in cps:
      cp.wait()
    base = wid * (SUBS * CHUNK) + k * CHUNK
    osl = pl.ds(base, CHUNK)
    pltpu.sync_copy(r_ui, o_ui.at[osl])
    pltpu.sync_copy(r_up, o_up.at[osl])
    pltpu.sync_copy(r_pi, o_pi.at[osl])
    pltpu.sync_copy(r_pp, o_pp.at[osl])
    pltpu.sync_copy(r_ni, o_ni.at[osl])
    pltpu.sync_copy(r_np, o_np.at[osl])
    return 0

  lax.fori_loop(0, SUBS, sub, 0)


_gather = pl.kernel(
    _gather_body,
    out_type=tuple(jax.ShapeDtypeStruct((N_OCC, D), jnp.float32)
                   for _ in range(6)),
    mesh=_mesh,
    compiler_params=pltpu.CompilerParams(use_tc_tiling_on_sc=False),
    scratch_types=[
        pltpu.VMEM((CH_ROWS, 128), jnp.int32),
        pltpu.VMEM((CH_ROWS, 128), jnp.int32),
        pltpu.VMEM((CH_ROWS, 128), jnp.int32),
        pltpu.VMEM((CHUNK, D), jnp.float32),
        pltpu.VMEM((CHUNK, D), jnp.float32),
        pltpu.VMEM((CHUNK, D), jnp.float32),
        pltpu.VMEM((CHUNK, D), jnp.float32),
        pltpu.VMEM((CHUNK, D), jnp.float32),
        pltpu.VMEM((CHUNK, D), jnp.float32),
        pltpu.SemaphoreType.DMA,
    ],
)


def _disc_body(uidx, pidx, nidx, ru, rp, rn,
               counts, zbuf, onesb, idxb, cntb, recb):
  c = lax.axis_index("c")
  s = lax.axis_index("s")

  def fill(i, _):
    zbuf[pl.ds(i * LN, LN)] = jnp.zeros((LN,), jnp.float32)
    return 0

  lax.fori_loop(0, ZB // LN, fill, 0)

  def ones_fill(i, _):
    onesb[pl.ds(i * LN, LN)] = jnp.ones((LN,), jnp.float32)
    return 0

  lax.fori_loop(0, 128 // LN, ones_fill, 0)

  def zero(i, _):
    pltpu.sync_copy(zbuf, counts.at[pl.ds(s * (NTAB // NS) + i * ZB, ZB)])
    return 0

  lax.fori_loop(0, NTAB // NS // ZB, zero, 0)
  plsc.subcore_barrier()

  def scat_rows(idx_hbm, r0):
    def batch(b, _):
      pltpu.sync_copy(idx_hbm.at[pl.ds(r0 + b * RB, RB)], idxb)
      for j in range(RB):
        pltpu.sync_copy(onesb, counts.at[idxb.at[j]], add=True)
      return 0

    lax.fori_loop(0, ROWS_PER_TILE // RB, batch, 0)

  @pl.when(c == 0)
  def _():
    scat_rows(uidx, s * ROWS_PER_TILE)

  @pl.when(c == 1)
  def _():
    scat_rows(pidx, s * ROWS_PER_TILE)
    scat_rows(nidx, s * ROWS_PER_TILE)

  plsc.subcore_barrier()

  def rec_rows(idx_hbm, out_hbm, r0):
    def batch(b, _):
      pltpu.sync_copy(idx_hbm.at[pl.ds(r0 + b * RB, RB)], idxb)
      for j in range(RB):
        pltpu.sync_copy(counts.at[idxb.at[j]], cntb.at[pl.ds(j * 128, 128)])

      def vec(k, _):
        o = pl.ds(k * LN, LN)
        recb[o] = 1.0 / cntb[o]
        return 0

      lax.fori_loop(0, RB * 128 // LN, vec, 0)
      pltpu.sync_copy(recb, out_hbm.at[pl.ds((r0 + b * RB) * 128, RB * 128)])
      return 0

    lax.fori_loop(0, ROWS_PER_TILE // RB, batch, 0)

  @pl.when(c == 0)
  def _():
    rec_rows(uidx, ru, s * ROWS_PER_TILE)

  @pl.when(c == 1)
  def _():
    rec_rows(pidx, rp, s * ROWS_PER_TILE)
    rec_rows(nidx, rn, s * ROWS_PER_TILE)


_disc = pl.kernel(
    _disc_body,
    out_type=tuple(jax.ShapeDtypeStruct((N_OCC,), jnp.float32)
                   for _ in range(3)),
    mesh=_mesh,
    compiler_params=pltpu.CompilerParams(use_tc_tiling_on_sc=False),
    scratch_types=[
        pltpu.VMEM_SHARED((NTAB,), jnp.float32),
        pltpu.VMEM((ZB,), jnp.float32),
        pltpu.VMEM((128,), jnp.float32),
        pltpu.VMEM((RB, 128), jnp.int32),
        pltpu.VMEM((RB * 128,), jnp.float32),
        pltpu.VMEM((RB * 128,), jnp.float32),
    ],
)


def _score_body(ui_ref, up_ref, pi_ref, pp_ref, ni_ref, nq_ref, out_ref):
  ui = ui_ref[...]
  up = up_ref[...]
  pi = pi_ref[...]
  pp = pp_ref[...]
  ni = ni_ref[...]
  nq = nq_ref[...]
  l_id = jax.lax.broadcasted_iota(jnp.int32, (128, 8), 0)
  j_id = jax.lax.broadcasted_iota(jnp.int32, (128, 8), 1)
  g = (l_id // D == j_id).astype(jnp.float32)

  def gsum(z):
    return jnp.dot(z, g, preferred_element_type=jnp.float32)

  di = gsum(ui * (pi - ni))
  dp = gsum(up * (pp - nq))
  su = gsum((ui - up) ** 2)
  sp = gsum((pi - pp) ** 2)
  sn = gsum((ni - nq) ** 2)
  pad = jnp.zeros((di.shape[0], 128 - 5 * 8), jnp.float32)
  out_ref[...] = jnp.concatenate([di, dp, su, sp, sn, pad], axis=1)


_score = pl.pallas_call(
    _score_body,
    grid=(TGRID,),
    in_specs=[pl.BlockSpec((BLK * D // 128, 128), lambda k: (k, 0))
              for _ in range(6)],
    out_specs=pl.BlockSpec((BLK * D // 128, 128), lambda k: (k, 0)),
    out_shape=jax.ShapeDtypeStruct((N_OCC * D // 128, 128), jnp.float32),
    compiler_params=pltpu.CompilerParams(
        dimension_semantics=("parallel",)),
)


def _loss_body(di_ref, dp_ref, su_ref, sp_ref, sn_ref,
               m_ref, ru_ref, rp_ref, rn_ref, out_ref):
  a = di_ref[...]
  b = dp_ref[...]
  m = m_ref[...]
  ru = ru_ref[...]
  rp = rp_ref[...]
  rn = rn_ref[...]
  eps = 1e-10
  n = float(N_OCC)

  def lg(x):
    return jnp.log(jax.nn.sigmoid(x) + eps)

  loss_int = -jnp.sum(m * lg(a)) / n
  loss_pop = -jnp.sum(m * lg(-b) + (1.0 - m) * lg(b)) / n
  loss_total = -jnp.sum(lg(a + b)) / n
  t_u = jnp.sum(su_ref[...] * ru)
  n_u = jnp.sum(ru)
  t_i = jnp.sum(sp_ref[...] * rp + sn_ref[...] * rn)
  n_i = jnp.sum(rp + rn)
  disc = t_u / (n_u * D) + t_i / (n_i * D)
  loss = 0.1 * loss_int + 0.1 * loss_pop + loss_total - 0.01 * disc
  out_ref[...] = loss.reshape(1, 1)


_loss = pl.pallas_call(
    _loss_body,
    out_shape=jax.ShapeDtypeStruct((1, 1), jnp.float32),
)


@jax.jit
def kernel(user, item_p, item_n, mask, users_int_w, users_pop_w,
           items_int_w, items_pop_w):
  u = user.astype(jnp.int32).reshape(IDX_ROWS, 128)
  p = item_p.astype(jnp.int32).reshape(IDX_ROWS, 128)
  q = item_n.astype(jnp.int32).reshape(IDX_ROWS, 128)
  rows = _gather(users_int_w, users_pop_w, items_int_w, items_pop_w, u, p, q)
  ru, rp, rn = _disc(u, p, q)
  packed = _score(*(r.reshape(N_OCC * D // 128, 128) for r in rows))

  def unpack(i):
    return jax.lax.slice(packed, (0, 8 * i), (N_OCC * D // 128, 8 * i + 8)
                         ).reshape(IDX_ROWS, 128)

  maskf = mask.reshape(-1).astype(jnp.float32).reshape(IDX_ROWS, 128)
  loss = _loss(unpack(0), unpack(1), unpack(2), unpack(3), unpack(4),
               maskf, ru.reshape(IDX_ROWS, 128), rp.reshape(IDX_ROWS, 128),
               rn.reshape(IDX_ROWS, 128))
  return loss[0, 0]


# speedup vs baseline: 2.0043x; 1.0175x over previous
import jax
import jax.numpy as jnp
from jax import lax
from jax.experimental import pallas as pl
from jax.experimental.pallas import tpu as pltpu
from jax.experimental.pallas import tpu_sc as plsc

D = 16
LN = 16
NC, NS = 2, 16
NW = NC * NS
N_OCC = 16384 * 20
IDX_ROWS = N_OCC // 128
CHUNK = 512
SUBS = N_OCC // NW // CHUNK
CH_ROWS = CHUNK // 128
NTAB = 1 << 20
ZB = 4096
GRPS = CHUNK // LN
ROWS_PER_TILE = IDX_ROWS // NS
RB = 8
BLK = 4096
TGRID = N_OCC // BLK

_mesh = plsc.VectorSubcoreMesh(core_axis_name="c", subcore_axis_name="s")


def _scores_body(uiw, upw, iiw, ipw, uidx, pidx, nidx,
                 d_int, d_pop, squ, sqp, sqn,
                 idx_u, idx_p, idx_n,
                 r_ui, r_up, r_pi, r_pp, r_ni, r_np,
                 b_dint, b_dpop, b_squ, b_sqp, b_sqn, sem):
  c = lax.axis_index("c")
  s = lax.axis_index("s")
  wid = c * NS + s

  def sub(k, _):
    row0 = wid * (SUBS * CH_ROWS) + k * CH_ROWS
    pltpu.sync_copy(uidx.at[pl.ds(row0, CH_ROWS)], idx_u)
    pltpu.sync_copy(pidx.at[pl.ds(row0, CH_ROWS)], idx_p)
    pltpu.sync_copy(nidx.at[pl.ds(row0, CH_ROWS)], idx_n)
    cps = []
    for j in range(CH_ROWS):
      sl = pl.ds(j * 128, 128)
      cps.append(pltpu.async_copy(uiw.at[idx_u.at[j]], r_ui.at[sl], sem))
      cps.append(pltpu.async_copy(upw.at[idx_u.at[j]], r_up.at[sl], sem))
      cps.append(pltpu.async_copy(iiw.at[idx_p.at[j]], r_pi.at[sl], sem))
      cps.append(pltpu.async_copy(ipw.at[idx_p.at[j]], r_pp.at[sl], sem))
      cps.append(pltpu.async_copy(iiw.at[idx_n.at[j]], r_ni.at[sl], sem))
      cps.append(pltpu.async_copy(ipw.at[idx_n.at[j]], r_np.at[sl], sem))
    for cp in cps:
      cp.wait()

    def grp(g, _):
      rid = lax.iota(jnp.int32, LN) + g * LN
      z = jnp.zeros((LN,), jnp.float32)
      a_di, a_dp = z, z
      a_su, a_sp, a_sn = z, z, z
      for d in range(D):
        col = jnp.full((LN,), d, jnp.int32)
        ui = plsc.load_gather(r_ui, [rid, col])
        up = plsc.load_gather(r_up, [rid, col])
        pi = plsc.load_gather(r_pi, [rid, col])
        pp = plsc.load_gather(r_pp, [rid, col])
        ni = plsc.load_gather(r_ni, [rid, col])
        nq = plsc.load_gather(r_np, [rid, col])
        a_di = a_di + ui * (pi - ni)
        a_dp = a_dp + up * (pp - nq)
        du = ui - up
        a_su = a_su + du * du
        dp_ = pi - pp
        a_sp = a_sp + dp_ * dp_
        dn = ni - nq
        a_sn = a_sn + dn * dn
      o = pl.ds(g * LN, LN)
      b_dint[o] = a_di
      b_dpop[o] = a_dp
      b_squ[o] = a_su
      b_sqp[o] = a_sp
      b_sqn[o] = a_sn
      return 0

    lax.fori_loop(0, GRPS, grp, 0)
    base = wid * (SUBS * CHUNK) + k * CHUNK
    osl = pl.ds(base, CHUNK)
    pltpu.sync_copy(b_dint, d_int.at[osl])
    pltpu.sync_copy(b_dpop, d_pop.at[osl])
    pltpu.sync_copy(b_squ, squ.at[osl])
    pltpu.sync_copy(b_sqp, sqp.at[osl])
    pltpu.sync_copy(b_sqn, sqn.at[osl])
    return 0

  lax.fori_loop(0, SUBS, sub, 0)


_scores = pl.kernel(
    _scores_body,
    out_type=tuple(jax.ShapeDtypeStruct((N_OCC,), jnp.float32)
                   for _ in range(5)),
    mesh=_mesh,
    compiler_params=pltpu.CompilerParams(use_tc_tiling_on_sc=False,
                                         needs_layout_passes=False),
    scratch_types=[
        pltpu.VMEM((CH_ROWS, 128), jnp.int32),
        pltpu.VMEM((CH_ROWS, 128), jnp.int32),
        pltpu.VMEM((CH_ROWS, 128), jnp.int32),
        pltpu.VMEM((CHUNK, D), jnp.float32),
        pltpu.VMEM((CHUNK, D), jnp.float32),
        pltpu.VMEM((CHUNK, D), jnp.float32),
        pltpu.VMEM((CHUNK, D), jnp.float32),
        pltpu.VMEM((CHUNK, D), jnp.float32),
        pltpu.VMEM((CHUNK, D), jnp.float32),
        pltpu.VMEM((CHUNK,), jnp.float32),
        pltpu.VMEM((CHUNK,), jnp.float32),
        pltpu.VMEM((CHUNK,), jnp.float32),
        pltpu.VMEM((CHUNK,), jnp.float32),
        pltpu.VMEM((CHUNK,), jnp.float32),
        pltpu.SemaphoreType.DMA,
    ],
)


def _disc_body(uidx, pidx, nidx, ru, rp, rn,
               counts, zbuf, onesb, idxb, cntb, recb):
  c = lax.axis_index("c")
  s = lax.axis_index("s")

  def fill(i, _):
    zbuf[pl.ds(i * LN, LN)] = jnp.zeros((LN,), jnp.float32)
    return 0

  lax.fori_loop(0, ZB // LN, fill, 0)

  def ones_fill(i, _):
    onesb[pl.ds(i * LN, LN)] = jnp.ones((LN,), jnp.float32)
    return 0

  lax.fori_loop(0, 128 // LN, ones_fill, 0)

  def zero(i, _):
    pltpu.sync_copy(zbuf, counts.at[pl.ds(s * (NTAB // NS) + i * ZB, ZB)])
    return 0

  lax.fori_loop(0, NTAB // NS // ZB, zero, 0)
  plsc.subcore_barrier()

  def scat_rows(idx_hbm, r0):
    def batch(b, _):
      pltpu.sync_copy(idx_hbm.at[pl.ds(r0 + b * RB, RB)], idxb)
      for j in range(RB):
        pltpu.sync_copy(onesb, counts.at[idxb.at[j]], add=True)
      return 0

    lax.fori_loop(0, ROWS_PER_TILE // RB, batch, 0)

  @pl.when(c == 0)
  def _():
    scat_rows(uidx, s * ROWS_PER_TILE)

  @pl.when(c == 1)
  def _():
    scat_rows(pidx, s * ROWS_PER_TILE)
    scat_rows(nidx, s * ROWS_PER_TILE)

  plsc.subcore_barrier()

  def rec_rows(idx_hbm, out_hbm, r0):
    def batch(b, _):
      pltpu.sync_copy(idx_hbm.at[pl.ds(r0 + b * RB, RB)], idxb)
      for j in range(RB):
        pltpu.sync_copy(counts.at[idxb.at[j]], cntb.at[pl.ds(j * 128, 128)])

      def vec(k, _):
        o = pl.ds(k * LN, LN)
        recb[o] = 1.0 / cntb[o]
        return 0

      lax.fori_loop(0, RB * 128 // LN, vec, 0)
      pltpu.sync_copy(recb, out_hbm.at[pl.ds((r0 + b * RB) * 128, RB * 128)])
      return 0

    lax.fori_loop(0, ROWS_PER_TILE // RB, batch, 0)

  @pl.when(c == 0)
  def _():
    rec_rows(uidx, ru, s * ROWS_PER_TILE)

  @pl.when(c == 1)
  def _():
    rec_rows(pidx, rp, s * ROWS_PER_TILE)
    rec_rows(nidx, rn, s * ROWS_PER_TILE)


_disc = pl.kernel(
    _disc_body,
    out_type=tuple(jax.ShapeDtypeStruct((N_OCC,), jnp.float32)
                   for _ in range(3)),
    mesh=_mesh,
    compiler_params=pltpu.CompilerParams(use_tc_tiling_on_sc=False),
    scratch_types=[
        pltpu.VMEM_SHARED((NTAB,), jnp.float32),
        pltpu.VMEM((ZB,), jnp.float32),
        pltpu.VMEM((128,), jnp.float32),
        pltpu.VMEM((RB, 128), jnp.int32),
        pltpu.VMEM((RB * 128,), jnp.float32),
        pltpu.VMEM((RB * 128,), jnp.float32),
    ],
)


def _loss_body(di_ref, dp_ref, su_ref, sp_ref, sn_ref,
               m_ref, ru_ref, rp_ref, rn_ref, out_ref):
  a = di_ref[...]
  b = dp_ref[...]
  m = m_ref[...]
  ru = ru_ref[...]
  rp = rp_ref[...]
  rn = rn_ref[...]
  eps = 1e-10
  n = float(N_OCC)

  def lg(x):
    return jnp.log(jax.nn.sigmoid(x) + eps)

  loss_int = -jnp.sum(m * lg(a)) / n
  loss_pop = -jnp.sum(m * lg(-b) + (1.0 - m) * lg(b)) / n
  loss_total = -jnp.sum(lg(a + b)) / n
  t_u = jnp.sum(su_ref[...] * ru)
  n_u = jnp.sum(ru)
  t_i = jnp.sum(sp_ref[...] * rp + sn_ref[...] * rn)
  n_i = jnp.sum(rp + rn)
  disc = t_u / (n_u * D) + t_i / (n_i * D)
  loss = 0.1 * loss_int + 0.1 * loss_pop + loss_total - 0.01 * disc
  out_ref[...] = loss.reshape(1, 1)


_loss = pl.pallas_call(
    _loss_body,
    out_shape=jax.ShapeDtypeStruct((1, 1), jnp.float32),
)


@jax.jit
def kernel(user, item_p, item_n, mask, users_int_w, users_pop_w,
           items_int_w, items_pop_w):
  u = user.astype(jnp.int32).reshape(IDX_ROWS, 128)
  p = item_p.astype(jnp.int32).reshape(IDX_ROWS, 128)
  q = item_n.astype(jnp.int32).reshape(IDX_ROWS, 128)
  d_int, d_pop, squ, sqp, sqn = _scores(
      users_int_w, users_pop_w, items_int_w, items_pop_w, u, p, q)
  ru, rp, rn = _disc(u, p, q)
  maskf = mask.reshape(-1).astype(jnp.float32).reshape(IDX_ROWS, 128)

  def occ(x):
    return x.reshape(IDX_ROWS, 128)

  loss = _loss(occ(d_int), occ(d_pop), occ(squ), occ(sqp), occ(sqn),
               maskf, occ(ru), occ(rp), occ(rn))
  return loss[0, 0]


# speedup vs baseline: 2.0922x; 1.0438x over previous
import jax
import jax.numpy as jnp
from jax import lax
from jax.experimental import pallas as pl
from jax.experimental.pallas import tpu as pltpu
from jax.experimental.pallas import tpu_sc as plsc

D = 16
LN = 16
NC, NS = 2, 16
NW = NC * NS
N_OCC = 16384 * 20
IDX_ROWS = N_OCC // 128
CHUNK = 512
SUBS = N_OCC // NW // CHUNK
CH_ROWS = CHUNK // 128
NTAB = 1 << 20
ZB = 4096
GRPS = CHUNK // LN
ROWS_PER_TILE = IDX_ROWS // NS
RB = 8
BLK = 4096
TGRID = N_OCC // BLK

_mesh = plsc.VectorSubcoreMesh(core_axis_name="c", subcore_axis_name="s")


def _scores_body(uiw, upw, iiw, ipw, uidx, pidx, nidx,
                 d_int, d_pop, squ, sqp, sqn,
                 idx_u0, idx_p0, idx_n0, idx_u1, idx_p1, idx_n1,
                 ui0, up0, pi0, pp0, ni0, np0,
                 ui1, up1, pi1, pp1, ni1, np1,
                 b_dint, b_dpop, b_squ, b_sqp, b_sqn, sem0, sem1):
  c = lax.axis_index("c")
  s = lax.axis_index("s")
  wid = c * NS + s
  idxs = ((idx_u0, idx_p0, idx_n0), (idx_u1, idx_p1, idx_n1))
  bufs = ((ui0, up0, pi0, pp0, ni0, np0), (ui1, up1, pi1, pp1, ni1, np1))
  sems = (sem0, sem1)

  def fire(k, par):
    iu, ip, iq = idxs[par]
    bu = bufs[par]
    sem = sems[par]

    @pl.when(k < SUBS)
    def _():
      row0 = wid * (SUBS * CH_ROWS) + k * CH_ROWS
      pltpu.sync_copy(uidx.at[pl.ds(row0, CH_ROWS)], iu)
      pltpu.sync_copy(pidx.at[pl.ds(row0, CH_ROWS)], ip)
      pltpu.sync_copy(nidx.at[pl.ds(row0, CH_ROWS)], iq)
      for j in range(CH_ROWS):
        sl = pl.ds(j * 128, 128)
        pltpu.async_copy(uiw.at[iu.at[j]], bu[0].at[sl], sem)
        pltpu.async_copy(upw.at[iu.at[j]], bu[1].at[sl], sem)
        pltpu.async_copy(iiw.at[ip.at[j]], bu[2].at[sl], sem)
        pltpu.async_copy(ipw.at[ip.at[j]], bu[3].at[sl], sem)
        pltpu.async_copy(iiw.at[iq.at[j]], bu[4].at[sl], sem)
        pltpu.async_copy(ipw.at[iq.at[j]], bu[5].at[sl], sem)

  def drain(par):
    for t in range(6):
      pltpu.make_async_copy(uiw.at[pl.ds(0, CHUNK)], bufs[par][t],
                            sems[par]).wait()

  def compute(k, par):
    r_ui, r_up, r_pi, r_pp, r_ni, r_np = bufs[par]

    def grp(g, _):
      rid = lax.iota(jnp.int32, LN) + g * LN
      z = jnp.zeros((LN,), jnp.float32)
      a_di, a_dp = z, z
      a_su, a_sp, a_sn = z, z, z
      for d in range(D):
        col = jnp.full((LN,), d, jnp.int32)
        ui = plsc.load_gather(r_ui, [rid, col])
        up = plsc.load_gather(r_up, [rid, col])
        pi = plsc.load_gather(r_pi, [rid, col])
        pp = plsc.load_gather(r_pp, [rid, col])
        ni = plsc.load_gather(r_ni, [rid, col])
        nq = plsc.load_gather(r_np, [rid, col])
        a_di = a_di + ui * (pi - ni)
        a_dp = a_dp + up * (pp - nq)
        du = ui - up
        a_su = a_su + du * du
        dp_ = pi - pp
        a_sp = a_sp + dp_ * dp_
        dn = ni - nq
        a_sn = a_sn + dn * dn
      o = pl.ds(g * LN, LN)
      b_dint[o] = a_di
      b_dpop[o] = a_dp
      b_squ[o] = a_su
      b_sqp[o] = a_sp
      b_sqn[o] = a_sn
      return 0

    lax.fori_loop(0, GRPS, grp, 0)
    base = wid * (SUBS * CHUNK) + k * CHUNK
    osl = pl.ds(base, CHUNK)
    pltpu.sync_copy(b_dint, d_int.at[osl])
    pltpu.sync_copy(b_dpop, d_pop.at[osl])
    pltpu.sync_copy(b_squ, squ.at[osl])
    pltpu.sync_copy(b_sqp, sqp.at[osl])
    pltpu.sync_copy(b_sqn, sqn.at[osl])

  fire(0, 0)

  def pair(i, _):
    k0 = i * 2
    drain(0)
    fire(k0 + 1, 1)
    compute(k0, 0)
    drain(1)
    fire(k0 + 2, 0)
    compute(k0 + 1, 1)
    return 0

  lax.fori_loop(0, SUBS // 2, pair, 0)


_scores = pl.kernel(
    _scores_body,
    out_type=tuple(jax.ShapeDtypeStruct((N_OCC,), jnp.float32)
                   for _ in range(5)),
    mesh=_mesh,
    compiler_params=pltpu.CompilerParams(use_tc_tiling_on_sc=False,
                                         needs_layout_passes=False),
    scratch_types=(
        [pltpu.VMEM((CH_ROWS, 128), jnp.int32) for _ in range(6)]
        + [pltpu.VMEM((CHUNK, D), jnp.float32) for _ in range(12)]
        + [pltpu.VMEM((CHUNK,), jnp.float32) for _ in range(5)]
        + [pltpu.SemaphoreType.DMA, pltpu.SemaphoreType.DMA]
    ),
)


def _disc_body(uidx, pidx, nidx, ru, rp, rn,
               counts, zbuf, onesb, idxb, cntb, recb):
  c = lax.axis_index("c")
  s = lax.axis_index("s")

  def fill(i, _):
    zbuf[pl.ds(i * LN, LN)] = jnp.zeros((LN,), jnp.float32)
    return 0

  lax.fori_loop(0, ZB // LN, fill, 0)

  def ones_fill(i, _):
    onesb[pl.ds(i * LN, LN)] = jnp.ones((LN,), jnp.float32)
    return 0

  lax.fori_loop(0, 128 // LN, ones_fill, 0)

  def zero(i, _):
    pltpu.sync_copy(zbuf, counts.at[pl.ds(s * (NTAB // NS) + i * ZB, ZB)])
    return 0

  lax.fori_loop(0, NTAB // NS // ZB, zero, 0)
  plsc.subcore_barrier()

  def scat_rows(idx_hbm, r0):
    def batch(b, _):
      pltpu.sync_copy(idx_hbm.at[pl.ds(r0 + b * RB, RB)], idxb)
      for j in range(RB):
        pltpu.sync_copy(onesb, counts.at[idxb.at[j]], add=True)
      return 0

    lax.fori_loop(0, ROWS_PER_TILE // RB, batch, 0)

  @pl.when(c == 0)
  def _():
    scat_rows(uidx, s * ROWS_PER_TILE)

  @pl.when(c == 1)
  def _():
    scat_rows(pidx, s * ROWS_PER_TILE)
    scat_rows(nidx, s * ROWS_PER_TILE)

  plsc.subcore_barrier()

  def rec_rows(idx_hbm, out_hbm, r0):
    def batch(b, _):
      pltpu.sync_copy(idx_hbm.at[pl.ds(r0 + b * RB, RB)], idxb)
      for j in range(RB):
        pltpu.sync_copy(counts.at[idxb.at[j]], cntb.at[pl.ds(j * 128, 128)])

      def vec(k, _):
        o = pl.ds(k * LN, LN)
        recb[o] = 1.0 / cntb[o]
        return 0

      lax.fori_loop(0, RB * 128 // LN, vec, 0)
      pltpu.sync_copy(recb, out_hbm.at[pl.ds((r0 + b * RB) * 128, RB * 128)])
      return 0

    lax.fori_loop(0, ROWS_PER_TILE // RB, batch, 0)

  @pl.when(c == 0)
  def _():
    rec_rows(uidx, ru, s * ROWS_PER_TILE)

  @pl.when(c == 1)
  def _():
    rec_rows(pidx, rp, s * ROWS_PER_TILE)
    rec_rows(nidx, rn, s * ROWS_PER_TILE)


_disc = pl.kernel(
    _disc_body,
    out_type=tuple(jax.ShapeDtypeStruct((N_OCC,), jnp.float32)
                   for _ in range(3)),
    mesh=_mesh,
    compiler_params=pltpu.CompilerParams(use_tc_tiling_on_sc=False),
    scratch_types=[
        pltpu.VMEM_SHARED((NTAB,), jnp.float32),
        pltpu.VMEM((ZB,), jnp.float32),
        pltpu.VMEM((128,), jnp.float32),
        pltpu.VMEM((RB, 128), jnp.int32),
        pltpu.VMEM((RB * 128,), jnp.float32),
        pltpu.VMEM((RB * 128,), jnp.float32),
    ],
)


def _loss_body(di_ref, dp_ref, su_ref, sp_ref, sn_ref,
               m_ref, ru_ref, rp_ref, rn_ref, out_ref):
  a = di_ref[...]
  b = dp_ref[...]
  m = m_ref[...]
  ru = ru_ref[...]
  rp = rp_ref[...]
  rn = rn_ref[...]
  eps = 1e-10
  n = float(N_OCC)

  def lg(x):
    return jnp.log(jax.nn.sigmoid(x) + eps)

  loss_int = -jnp.sum(m * lg(a)) / n
  loss_pop = -jnp.sum(m * lg(-b) + (1.0 - m) * lg(b)) / n
  loss_total = -jnp.sum(lg(a + b)) / n
  t_u = jnp.sum(su_ref[...] * ru)
  n_u = jnp.sum(ru)
  t_i = jnp.sum(sp_ref[...] * rp + sn_ref[...] * rn)
  n_i = jnp.sum(rp + rn)
  disc = t_u / (n_u * D) + t_i / (n_i * D)
  loss = 0.1 * loss_int + 0.1 * loss_pop + loss_total - 0.01 * disc
  out_ref[...] = loss.reshape(1, 1)


_loss = pl.pallas_call(
    _loss_body,
    out_shape=jax.ShapeDtypeStruct((1, 1), jnp.float32),
)


@jax.jit
def kernel(user, item_p, item_n, mask, users_int_w, users_pop_w,
           items_int_w, items_pop_w):
  u = user.astype(jnp.int32).reshape(IDX_ROWS, 128)
  p = item_p.astype(jnp.int32).reshape(IDX_ROWS, 128)
  q = item_n.astype(jnp.int32).reshape(IDX_ROWS, 128)
  d_int, d_pop, squ, sqp, sqn = _scores(
      users_int_w, users_pop_w, items_int_w, items_pop_w, u, p, q)
  ru, rp, rn = _disc(u, p, q)
  maskf = mask.reshape(-1).astype(jnp.float32).reshape(IDX_ROWS, 128)

  def occ(x):
    return x.reshape(IDX_ROWS, 128)

  loss = _loss(occ(d_int), occ(d_pop), occ(squ), occ(sqp), occ(sqn),
               maskf, occ(ru), occ(rp), occ(rn))
  return loss[0, 0]
